# baseline (speedup 1.0000x reference)
import functools

import numpy as np

import jax
import jax.numpy as jnp
from jax import lax
from jax.experimental import pallas as pl
from jax.experimental.pallas import tpu as pltpu
from jax.experimental.pallas import tpu_sc as plsc

NC = 2
NS = 16
L = 16
NW = NC * NS
N = 4194304
D = 65536
PAD = 128
SHARD = N // NW
CH_A = 16384
CH_C = 8192
BT = D // NS
BIG = 0x7FFFFFFF

_mesh = plsc.VectorSubcoreMesh(core_axis_name="c", subcore_axis_name="s")


def _loop(n, body, init=0, unroll=1):
    assert n % unroll == 0

    def wrapped(i, carry):
        for u in range(unroll):
            carry = body(i * np.int32(unroll) + np.int32(u), carry)
        return carry

    return lax.fori_loop(jnp.int32(0), jnp.int32(n // unroll), wrapped, init)


@functools.partial(
    pl.kernel,
    out_type=jax.ShapeDtypeStruct((NS, NW, BT), jnp.int32),
    mesh=_mesh,
    compiler_params=pltpu.CompilerParams(needs_layout_passes=False),
    scratch_types=[
        pltpu.VMEM((D,), jnp.int32),
        pltpu.VMEM((2, CH_A), jnp.int32),
        pltpu.SemaphoreType.DMA,
        pltpu.SemaphoreType.DMA,
        pltpu.SemaphoreType.DMA,
    ],
)
def _hist_kernel(x_hbm, out_hbm, hist, buf, sem0, sem1, wsem):
    c = lax.axis_index("c")
    s = lax.axis_index("s")
    wid = s * NC + c
    base = wid * SHARD
    sems = [sem0, sem1]

    zero = jnp.zeros((L,), jnp.int32)

    def zbody(j, carry):
        hist[pl.ds(j * L, L)] = zero
        return carry

    _loop(D // L, zbody, unroll=8)

    ones = jnp.ones((L,), jnp.int32)
    nch = SHARD // CH_A
    handles = [None, None]
    handles[0] = pltpu.async_copy(x_hbm.at[pl.ds(base, CH_A)], buf.at[np.int32(0)], sems[0])
    for ci in range(nch):
        b = ci & 1
        if ci + 1 < nch:
            nb = (ci + 1) & 1
            handles[nb] = pltpu.async_copy(
                x_hbm.at[pl.ds(base + (ci + 1) * CH_A, CH_A)], buf.at[np.int32(nb)], sems[nb]
            )
        handles[b].wait()

        def body(i, carry):
            v = buf[np.int32(b), pl.ds(i * L, L)]
            plsc.addupdate_scatter(hist, [v], ones)
            return carry

        _loop(CH_A // L, body)

    whs = []
    for t in range(NS):
        whs.append(pltpu.async_copy(
            hist.at[pl.ds(np.int32(t * BT), BT)], out_hbm.at[np.int32(t), wid], wsem
        ))
    for h in whs:
        h.wait()


@functools.partial(
    pl.kernel,
    out_type=(
        jax.ShapeDtypeStruct((D,), jnp.int32),
        jax.ShapeDtypeStruct((D,), jnp.int32),
        jax.ShapeDtypeStruct((D,), jnp.int32),
        jax.ShapeDtypeStruct((NS, L), jnp.int32),
    ),
    mesh=_mesh,
    compiler_params=pltpu.CompilerParams(needs_layout_passes=False),
    scratch_types=[
        pltpu.VMEM((BT,), jnp.int32),
        pltpu.VMEM((2, NW // 4, BT), jnp.int32),
        pltpu.VMEM((NS, L), jnp.int32),
        pltpu.VMEM((L,), jnp.int32),
        pltpu.VMEM((BT,), jnp.int32),
        pltpu.VMEM((BT,), jnp.int32),
        pltpu.VMEM((BT,), jnp.int32),
        pltpu.VMEM((BT // 128, 128), jnp.int32),
        pltpu.VMEM_SHARED((D + PAD,), jnp.int32),
        pltpu.VMEM_SHARED((D + PAD,), jnp.int32),
        pltpu.SemaphoreType.DMA,
        pltpu.SemaphoreType.DMA,
    ],
)
def _scan_kernel(hists_hbm, axs_hbm, y_hbm, cnt_hbm, rank_hbm, xch_hbm,
                 acc, tmp, exch, pub, yv, cv, rv, idx2, ysp, csp, sem0, sem1):
    c = lax.axis_index("c")
    s = lax.axis_index("s")

    @pl.when(c == 0)
    def _():
        sems = [sem0, sem1]
        base = s * BT
        WB = NW // 4
        handles = [None, None]
        handles[0] = pltpu.async_copy(
            hists_hbm.at[s, pl.ds(np.int32(0), WB)], tmp.at[np.int32(0)], sems[0]
        )
        for blk in range(4):
            b = blk & 1
            if blk + 1 < 4:
                nb = (blk + 1) & 1
                handles[nb] = pltpu.async_copy(
                    hists_hbm.at[s, pl.ds(np.int32((blk + 1) * WB), WB)],
                    tmp.at[np.int32(nb)], sems[nb]
                )
            handles[b].wait()
            for w in range(WB):
                if blk == 0 and w == 0:
                    def cbody(i, carry):
                        acc[pl.ds(i * L, L)] = tmp[np.int32(0), np.int32(0), pl.ds(i * L, L)]
                        return carry
                    _loop(BT // L, cbody, unroll=8)
                else:
                    def abody(i, carry, _b=b, _w=w):
                        sl = pl.ds(i * L, L)
                        acc[sl] = acc[sl] + tmp[np.int32(_b), np.int32(_w), sl]
                        return carry
                    _loop(BT // L, abody, unroll=8)

        lanes = jnp.arange(L, dtype=jnp.int32)

        def pbody(i, carry):
            cntv, mv = carry
            h = acc[pl.ds(i * L, L)]
            p = h > 0
            pi = jnp.where(p, jnp.int32(1), jnp.int32(0))
            vals = base + i * L + lanes
            mv = jnp.minimum(mv, jnp.where(p, vals, BIG))
            return cntv + pi, mv

        cntv, mv_vec = _loop(
            BT // L, pbody,
            (jnp.zeros((L,), jnp.int32), jnp.full((L,), BIG, jnp.int32)), unroll=4
        )
        cnt_t = jnp.sum(cntv, dtype=jnp.int32)
        minv_t = jnp.min(mv_vec)

        pltpu.sync_copy(axs_hbm, pub)
        ax = pub[...][0]

        pub[...] = jnp.where(
            lanes == 0, cnt_t, jnp.where(lanes == 1, minv_t, jnp.int32(0))
        )
        pltpu.sync_copy(pub, xch_hbm.at[s])
        plsc.subcore_barrier()
        pltpu.sync_copy(xch_hbm, exch)
        off = jnp.int32(0)
        vmin = jnp.int32(BIG)
        for r in range(NS):
            row = exch[np.int32(r)]
            off = off + jnp.where(np.int32(r) < s, row[0], jnp.int32(0))
            vmin = jnp.minimum(vmin, row[1])

        vminv = jnp.broadcast_to(vmin, (L,))
        zerov = jnp.zeros((L,), jnp.int32)

        def ibody(i, carry):
            yv[pl.ds(i * L, L)] = vminv
            cv[pl.ds(i * L, L)] = zerov
            return carry

        _loop(BT // L, ibody, unroll=4)
        pltpu.sync_copy(yv, ysp.at[pl.ds(base, BT)])
        pltpu.sync_copy(cv, csp.at[pl.ds(base, BT)])
        plsc.subcore_barrier()

        trash = jnp.int32(D)

        def make_body(jj):
            def sbody(k, off_run):
                j = np.int32(jj * 8) + k
                sl = pl.ds(j * L, L)
                h = acc[sl]
                p = h > 0
                pi = jnp.where(p, jnp.int32(1), jnp.int32(0))
                sc = jnp.cumsum(pi)
                rank_vec = off_run + sc - pi
                rv[sl] = rank_vec + ax
                yv[sl] = base + j * L + lanes
                cv[sl] = h
                idx2[np.int32(jj), pl.ds(k * L, L)] = jnp.where(p, rank_vec, trash)
                return off_run + sc[L - 1]
            return sbody

        off_run = off
        for jj in range(BT // 128):
            off_run = _loop(128 // L, make_body(jj), off_run)

        pltpu.sync_copy(rv, rank_hbm.at[pl.ds(base, BT)])

        shs = []
        for q in range(BT // 128):
            shs.append(pltpu.async_copy(
                yv.at[pl.ds(np.int32(q * 128), 128)], ysp.at[idx2.at[np.int32(q)]], sem0
            ))
            shs.append(pltpu.async_copy(
                cv.at[pl.ds(np.int32(q * 128), 128)], csp.at[idx2.at[np.int32(q)]], sem1
            ))
        for h in shs:
            h.wait()
        plsc.subcore_barrier()
        pltpu.sync_copy(ysp.at[pl.ds(base, BT)], y_hbm.at[pl.ds(base, BT)])
        pltpu.sync_copy(csp.at[pl.ds(base, BT)], cnt_hbm.at[pl.ds(base, BT)])


@functools.partial(
    pl.kernel,
    out_type=jax.ShapeDtypeStruct((N,), jnp.int32),
    mesh=_mesh,
    compiler_params=pltpu.CompilerParams(needs_layout_passes=False),
    scratch_types=[
        pltpu.VMEM((D,), jnp.int32),
        pltpu.VMEM((2, CH_C), jnp.int32),
        pltpu.VMEM((2, CH_C), jnp.int32),
        pltpu.SemaphoreType.DMA,
        pltpu.SemaphoreType.DMA,
        pltpu.SemaphoreType.DMA,
        pltpu.SemaphoreType.DMA,
        pltpu.SemaphoreType.DMA,
    ],
)
def _gather_kernel(x_hbm, rank_hbm, out_hbm, table, ibuf, obuf,
                   isem0, isem1, osem0, osem1, tsem):
    c = lax.axis_index("c")
    s = lax.axis_index("s")
    wid = s * NC + c
    base = wid * SHARD
    isems = [isem0, isem1]
    osems = [osem0, osem1]

    th = pltpu.async_copy(rank_hbm, table, tsem)

    ncc = SHARD // CH_C
    ih = [None, None]
    oh = [None, None]
    ih[0] = pltpu.async_copy(x_hbm.at[pl.ds(base, CH_C)], ibuf.at[np.int32(0)], isems[0])
    th.wait()
    for ci in range(ncc):
        b = ci & 1
        if ci + 1 < ncc:
            nb = (ci + 1) & 1
            ih[nb] = pltpu.async_copy(
                x_hbm.at[pl.ds(base + (ci + 1) * CH_C, CH_C)], ibuf.at[np.int32(nb)], isems[nb]
            )
        ih[b].wait()
        if oh[b] is not None:
            oh[b].wait()

        def gbody(i, carry):
            v = ibuf[np.int32(b), pl.ds(i * L, L)]
            obuf[np.int32(b), pl.ds(i * L, L)] = plsc.load_gather(table, [v])
            return carry

        _loop(CH_C // L, gbody)
        oh[b] = pltpu.async_copy(
            obuf.at[np.int32(b)], out_hbm.at[pl.ds(base + ci * CH_C, CH_C)], osems[b]
        )
    for b in range(2):
        if oh[b] is not None:
            oh[b].wait()


def kernel(x, axis):
    x32 = x.astype(jnp.int32)
    hists = _hist_kernel(x32)
    axs = jnp.full((L,), axis, jnp.int32)
    y_pad, cnt_pad, rank, _ = _scan_kernel(hists, axs)
    idx32 = _gather_kernel(x32, rank)
    y = y_pad.astype(jnp.int64)
    cnt = cnt_pad.astype(jnp.int64)
    idx = idx32.astype(jnp.uint32).astype(jnp.int64)
    return (y, idx, cnt)


# speedup vs baseline: 1.0037x; 1.0037x over previous
import functools

import numpy as np

import jax
import jax.numpy as jnp
from jax import lax
from jax.experimental import pallas as pl
from jax.experimental.pallas import tpu as pltpu
from jax.experimental.pallas import tpu_sc as plsc

NC = 2
NS = 16
L = 16
NW = NC * NS
N = 4194304
D = 65536
PAD = 128
SHARD = N // NW
CH_A = 16384
CH_C = 8192
BT = D // NS
BIG = 0x7FFFFFFF

_mesh = plsc.VectorSubcoreMesh(core_axis_name="c", subcore_axis_name="s")


def _loop(n, body, init=0, unroll=1):
    assert n % unroll == 0

    def wrapped(i, carry):
        for u in range(unroll):
            carry = body(i * np.int32(unroll) + np.int32(u), carry)
        return carry

    return lax.fori_loop(jnp.int32(0), jnp.int32(n // unroll), wrapped, init)


@functools.partial(
    pl.kernel,
    out_type=jax.ShapeDtypeStruct((NS, NW, BT), jnp.int32),
    mesh=_mesh,
    compiler_params=pltpu.CompilerParams(needs_layout_passes=False),
    scratch_types=[
        pltpu.VMEM((D,), jnp.int32),
        pltpu.VMEM((2, CH_A), jnp.int32),
        pltpu.SemaphoreType.DMA,
        pltpu.SemaphoreType.DMA,
        pltpu.SemaphoreType.DMA,
    ],
)
def _hist_kernel(x_hbm, out_hbm, hist, buf, sem0, sem1, wsem):
    c = lax.axis_index("c")
    s = lax.axis_index("s")
    wid = s * NC + c
    base = wid * SHARD
    sems = [sem0, sem1]

    zero = jnp.zeros((L,), jnp.int32)

    def zbody(j, carry):
        hist[pl.ds(j * L, L)] = zero
        return carry

    _loop(D // L, zbody, unroll=8)

    ones = jnp.ones((L,), jnp.int32)
    nch = SHARD // CH_A
    handles = [None, None]
    handles[0] = pltpu.async_copy(x_hbm.at[pl.ds(base, CH_A)], buf.at[np.int32(0)], sems[0])
    for ci in range(nch):
        b = ci & 1
        if ci + 1 < nch:
            nb = (ci + 1) & 1
            handles[nb] = pltpu.async_copy(
                x_hbm.at[pl.ds(base + (ci + 1) * CH_A, CH_A)], buf.at[np.int32(nb)], sems[nb]
            )
        handles[b].wait()

        def body(i, vcur):
            vnext = buf[np.int32(b), pl.ds(((i + 1) & np.int32(CH_A // L - 1)) * L, L)]
            plsc.addupdate_scatter(hist, [vcur], ones)
            return vnext

        _loop(CH_A // L, body, init=buf[np.int32(b), pl.ds(jnp.int32(0), L)])

    whs = []
    for t in range(NS):
        whs.append(pltpu.async_copy(
            hist.at[pl.ds(np.int32(t * BT), BT)], out_hbm.at[np.int32(t), wid], wsem
        ))
    for h in whs:
        h.wait()


@functools.partial(
    pl.kernel,
    out_type=(
        jax.ShapeDtypeStruct((D,), jnp.int32),
        jax.ShapeDtypeStruct((D,), jnp.int32),
        jax.ShapeDtypeStruct((D,), jnp.int32),
        jax.ShapeDtypeStruct((NS, L), jnp.int32),
    ),
    mesh=_mesh,
    compiler_params=pltpu.CompilerParams(needs_layout_passes=False),
    scratch_types=[
        pltpu.VMEM((BT,), jnp.int32),
        pltpu.VMEM((2, NW // 4, BT), jnp.int32),
        pltpu.VMEM((NS, L), jnp.int32),
        pltpu.VMEM((L,), jnp.int32),
        pltpu.VMEM((BT,), jnp.int32),
        pltpu.VMEM((BT,), jnp.int32),
        pltpu.VMEM((BT,), jnp.int32),
        pltpu.VMEM((BT // 128, 128), jnp.int32),
        pltpu.VMEM_SHARED((D + PAD,), jnp.int32),
        pltpu.VMEM_SHARED((D + PAD,), jnp.int32),
        pltpu.SemaphoreType.DMA,
        pltpu.SemaphoreType.DMA,
    ],
)
def _scan_kernel(hists_hbm, axs_hbm, y_hbm, cnt_hbm, rank_hbm, xch_hbm,
                 acc, tmp, exch, pub, yv, cv, rv, idx2, ysp, csp, sem0, sem1):
    c = lax.axis_index("c")
    s = lax.axis_index("s")

    @pl.when(c == 0)
    def _():
        sems = [sem0, sem1]
        base = s * BT
        WB = NW // 4
        handles = [None, None]
        handles[0] = pltpu.async_copy(
            hists_hbm.at[s, pl.ds(np.int32(0), WB)], tmp.at[np.int32(0)], sems[0]
        )
        for blk in range(4):
            b = blk & 1
            if blk + 1 < 4:
                nb = (blk + 1) & 1
                handles[nb] = pltpu.async_copy(
                    hists_hbm.at[s, pl.ds(np.int32((blk + 1) * WB), WB)],
                    tmp.at[np.int32(nb)], sems[nb]
                )
            handles[b].wait()
            for w in range(WB):
                if blk == 0 and w == 0:
                    def cbody(i, carry):
                        acc[pl.ds(i * L, L)] = tmp[np.int32(0), np.int32(0), pl.ds(i * L, L)]
                        return carry
                    _loop(BT // L, cbody, unroll=8)
                else:
                    def abody(i, carry, _b=b, _w=w):
                        sl = pl.ds(i * L, L)
                        acc[sl] = acc[sl] + tmp[np.int32(_b), np.int32(_w), sl]
                        return carry
                    _loop(BT // L, abody, unroll=8)

        lanes = jnp.arange(L, dtype=jnp.int32)

        def pbody(i, carry):
            cntv, mv = carry
            h = acc[pl.ds(i * L, L)]
            p = h > 0
            pi = jnp.where(p, jnp.int32(1), jnp.int32(0))
            vals = base + i * L + lanes
            mv = jnp.minimum(mv, jnp.where(p, vals, BIG))
            return cntv + pi, mv

        cntv, mv_vec = _loop(
            BT // L, pbody,
            (jnp.zeros((L,), jnp.int32), jnp.full((L,), BIG, jnp.int32)), unroll=4
        )
        cnt_t = jnp.sum(cntv, dtype=jnp.int32)
        minv_t = jnp.min(mv_vec)

        pltpu.sync_copy(axs_hbm, pub)
        ax = pub[...][0]

        pub[...] = jnp.where(
            lanes == 0, cnt_t, jnp.where(lanes == 1, minv_t, jnp.int32(0))
        )
        pltpu.sync_copy(pub, xch_hbm.at[s])
        plsc.subcore_barrier()
        pltpu.sync_copy(xch_hbm, exch)
        off = jnp.int32(0)
        vmin = jnp.int32(BIG)
        for r in range(NS):
            row = exch[np.int32(r)]
            off = off + jnp.where(np.int32(r) < s, row[0], jnp.int32(0))
            vmin = jnp.minimum(vmin, row[1])

        vminv = jnp.broadcast_to(vmin, (L,))
        zerov = jnp.zeros((L,), jnp.int32)

        def ibody(i, carry):
            yv[pl.ds(i * L, L)] = vminv
            cv[pl.ds(i * L, L)] = zerov
            return carry

        _loop(BT // L, ibody, unroll=4)
        pltpu.sync_copy(yv, ysp.at[pl.ds(base, BT)])
        pltpu.sync_copy(cv, csp.at[pl.ds(base, BT)])
        plsc.subcore_barrier()

        trash = jnp.int32(D)

        def make_body(jj):
            def sbody(k, off_run):
                j = np.int32(jj * 8) + k
                sl = pl.ds(j * L, L)
                h = acc[sl]
                p = h > 0
                pi = jnp.where(p, jnp.int32(1), jnp.int32(0))
                sc = jnp.cumsum(pi)
                rank_vec = off_run + sc - pi
                rv[sl] = rank_vec + ax
                yv[sl] = base + j * L + lanes
                cv[sl] = h
                idx2[np.int32(jj), pl.ds(k * L, L)] = jnp.where(p, rank_vec, trash)
                return off_run + sc[L - 1]
            return sbody

        off_run = off
        for jj in range(BT // 128):
            off_run = _loop(128 // L, make_body(jj), off_run)

        pltpu.sync_copy(rv, rank_hbm.at[pl.ds(base, BT)])

        shs = []
        for q in range(BT // 128):
            shs.append(pltpu.async_copy(
                yv.at[pl.ds(np.int32(q * 128), 128)], ysp.at[idx2.at[np.int32(q)]], sem0
            ))
            shs.append(pltpu.async_copy(
                cv.at[pl.ds(np.int32(q * 128), 128)], csp.at[idx2.at[np.int32(q)]], sem1
            ))
        for h in shs:
            h.wait()
        plsc.subcore_barrier()
        pltpu.sync_copy(ysp.at[pl.ds(base, BT)], y_hbm.at[pl.ds(base, BT)])
        pltpu.sync_copy(csp.at[pl.ds(base, BT)], cnt_hbm.at[pl.ds(base, BT)])


@functools.partial(
    pl.kernel,
    out_type=jax.ShapeDtypeStruct((N,), jnp.int32),
    mesh=_mesh,
    compiler_params=pltpu.CompilerParams(needs_layout_passes=False),
    scratch_types=[
        pltpu.VMEM((D,), jnp.int32),
        pltpu.VMEM((2, CH_C), jnp.int32),
        pltpu.VMEM((2, CH_C), jnp.int32),
        pltpu.SemaphoreType.DMA,
        pltpu.SemaphoreType.DMA,
        pltpu.SemaphoreType.DMA,
        pltpu.SemaphoreType.DMA,
        pltpu.SemaphoreType.DMA,
    ],
)
def _gather_kernel(x_hbm, rank_hbm, out_hbm, table, ibuf, obuf,
                   isem0, isem1, osem0, osem1, tsem):
    c = lax.axis_index("c")
    s = lax.axis_index("s")
    wid = s * NC + c
    base = wid * SHARD
    isems = [isem0, isem1]
    osems = [osem0, osem1]

    th = pltpu.async_copy(rank_hbm, table, tsem)

    ncc = SHARD // CH_C
    ih = [None, None]
    oh = [None, None]
    ih[0] = pltpu.async_copy(x_hbm.at[pl.ds(base, CH_C)], ibuf.at[np.int32(0)], isems[0])
    th.wait()
    for ci in range(ncc):
        b = ci & 1
        if ci + 1 < ncc:
            nb = (ci + 1) & 1
            ih[nb] = pltpu.async_copy(
                x_hbm.at[pl.ds(base + (ci + 1) * CH_C, CH_C)], ibuf.at[np.int32(nb)], isems[nb]
            )
        ih[b].wait()
        if oh[b] is not None:
            oh[b].wait()

        def gbody(i, vcur):
            vnext = ibuf[np.int32(b), pl.ds(((i + 1) & np.int32(CH_C // L - 1)) * L, L)]
            obuf[np.int32(b), pl.ds(i * L, L)] = plsc.load_gather(table, [vcur])
            return vnext

        _loop(CH_C // L, gbody, init=ibuf[np.int32(b), pl.ds(jnp.int32(0), L)])
        oh[b] = pltpu.async_copy(
            obuf.at[np.int32(b)], out_hbm.at[pl.ds(base + ci * CH_C, CH_C)], osems[b]
        )
    for b in range(2):
        if oh[b] is not None:
            oh[b].wait()


def kernel(x, axis):
    x32 = x.astype(jnp.int32)
    hists = _hist_kernel(x32)
    axs = jnp.full((L,), axis, jnp.int32)
    y_pad, cnt_pad, rank, _ = _scan_kernel(hists, axs)
    idx32 = _gather_kernel(x32, rank)
    y = y_pad.astype(jnp.int64)
    cnt = cnt_pad.astype(jnp.int64)
    idx = idx32.astype(jnp.uint32).astype(jnp.int64)
    return (y, idx, cnt)
